# Initial kernel scaffold; baseline (speedup 1.0000x reference)
import jax
import jax.numpy as jnp
from jax.experimental import pallas as pl


def kernel(X, mask, W1, b1, w2, b2, Wc, bc):
    raise NotImplementedError("write your pallas kernel here")


# speedup vs baseline: 1.0657x; 1.0657x over previous
import jax
import jax.numpy as jnp
from jax.experimental import pallas as pl
from jax.experimental.pallas import tpu as pltpu

FEAT = 512
ATT = 128


def _mil_kernel(x_ref, m_ref, w1_ref, b1_ref, w2_ref, b2_ref, wc_ref, bc_ref,
                out_ref):
    x = x_ref[0]
    h = jnp.tanh(
        jnp.dot(x, w1_ref[...], preferred_element_type=jnp.float32)
        + b1_ref[...])
    f = jnp.sum(h * w2_ref[...], axis=1, keepdims=True) + b2_ref[0, 0]
    c = jnp.sum(x * wc_ref[...], axis=1, keepdims=True)
    mf = m_ref[0]
    f = jnp.where(mf > 0, f, jnp.float32(-1e9))
    fmax = jnp.max(f, keepdims=True)
    e = jnp.exp(f - fmax)
    denom = jnp.sum(e, keepdims=True)
    num = jnp.sum(e * c, keepdims=True)
    out_ref[0] = num / denom + bc_ref[...]


def kernel(X, mask, W1, b1, w2, b2, Wc, bc):
    B, BAG, _ = X.shape
    mask_f = mask.astype(jnp.float32).reshape(B, BAG, 1)
    out = pl.pallas_call(
        _mil_kernel,
        grid=(B,),
        in_specs=[
            pl.BlockSpec((1, BAG, FEAT), lambda b: (b, 0, 0)),
            pl.BlockSpec((1, BAG, 1), lambda b: (b, 0, 0)),
            pl.BlockSpec((FEAT, ATT), lambda b: (0, 0)),
            pl.BlockSpec((1, ATT), lambda b: (0, 0)),
            pl.BlockSpec((1, ATT), lambda b: (0, 0)),
            pl.BlockSpec((1, 1), lambda b: (0, 0)),
            pl.BlockSpec((1, FEAT), lambda b: (0, 0)),
            pl.BlockSpec((1, 1), lambda b: (0, 0)),
        ],
        out_specs=pl.BlockSpec((1, 1, 1), lambda b: (b, 0, 0)),
        out_shape=jax.ShapeDtypeStruct((B, 1, 1), jnp.float32),
        compiler_params=pltpu.CompilerParams(
            dimension_semantics=("arbitrary",)),
    )(X, mask_f, W1, b1.reshape(1, ATT), w2.reshape(1, ATT),
      b2.reshape(1, 1), Wc.reshape(1, FEAT), bc.reshape(1, 1))
    return out[:, 0, 0]


# speedup vs baseline: 1.1410x; 1.0707x over previous
import jax
import jax.numpy as jnp
from jax.experimental import pallas as pl
from jax.experimental.pallas import tpu as pltpu

FEAT = 512
ATT = 128


def _mil_kernel(x_ref, m_ref, w1a_ref, b1_ref, w2_ref, b2_ref, bc_ref,
                out_ref):
    x = x_ref[0]
    pre = jnp.dot(x, w1a_ref[...], preferred_element_type=jnp.float32)
    h = jnp.tanh(pre[:, :ATT] + b1_ref[...])
    c = pre[:, ATT:ATT + 1]
    f = jnp.sum(h * w2_ref[...], axis=1, keepdims=True) + b2_ref[...]
    e = jnp.exp(f) * m_ref[0]
    denom = jnp.sum(e, keepdims=True)
    num = jnp.sum(e * c, keepdims=True)
    out_ref[0] = num / denom + bc_ref[...]


def kernel(X, mask, W1, b1, w2, b2, Wc, bc):
    B, BAG, _ = X.shape
    mask_f = mask.astype(jnp.float32).reshape(B, BAG, 1)
    w1aug = jnp.pad(jnp.concatenate([W1, Wc], axis=1),
                    ((0, 0), (0, ATT - 1)))
    out = pl.pallas_call(
        _mil_kernel,
        grid=(B,),
        in_specs=[
            pl.BlockSpec((1, BAG, FEAT), lambda b: (b, 0, 0)),
            pl.BlockSpec((1, BAG, 1), lambda b: (b, 0, 0)),
            pl.BlockSpec((FEAT, 2 * ATT), lambda b: (0, 0)),
            pl.BlockSpec((1, ATT), lambda b: (0, 0)),
            pl.BlockSpec((1, ATT), lambda b: (0, 0)),
            pl.BlockSpec((1, 1), lambda b: (0, 0)),
            pl.BlockSpec((1, 1), lambda b: (0, 0)),
        ],
        out_specs=pl.BlockSpec((1, 1, 1), lambda b: (b, 0, 0)),
        out_shape=jax.ShapeDtypeStruct((B, 1, 1), jnp.float32),
        compiler_params=pltpu.CompilerParams(
            dimension_semantics=("arbitrary",)),
    )(X, mask_f, w1aug, b1.reshape(1, ATT), w2.reshape(1, ATT),
      b2.reshape(1, 1), bc.reshape(1, 1))
    return out[:, 0, 0]
